# baseline (speedup 1.0000x reference)
import functools

import jax
import jax.numpy as jnp
from jax import lax
from jax.experimental import pallas as pl
from jax.experimental.pallas import tpu as pltpu
from jax.experimental.pallas import tpu_sc as plsc

_NUM_TAGS = 64
_HIGH = 5.0
_LOW = -5.0

_B, _S, _K = 256, 40, 30
_NPAIR = _B * _S

_NW = 32
_PAIRS_PER_W = 48
_SC_PAIRS = _NW * _PAIRS_PER_W
_CH = 8
_NCH = _PAIRS_PER_W // _CH
_NB = 2

_TC_PAIRS = _NPAIR - _SC_PAIRS
_TC_G = 512


@functools.partial(
    pl.kernel,
    out_type=jax.ShapeDtypeStruct((_SC_PAIRS, _K, _NUM_TAGS), jnp.float32),
    mesh=plsc.VectorSubcoreMesh(core_axis_name="c", subcore_axis_name="s"),
    scratch_types=[
        pltpu.VMEM((_PAIRS_PER_W, _K), jnp.int32),
        pltpu.VMEM((_CH, _K, _NUM_TAGS), jnp.float32),
        pltpu.VMEM((_CH, _K, _NUM_TAGS), jnp.float32),
        pltpu.SemaphoreType.DMA,
        pltpu.SemaphoreType.DMA,
    ],
)
def _score_expand_sc(labels_hbm, out_hbm, idx_v, buf0, buf1, sem0, sem1):
    bufs = (buf0, buf1)
    sems = (sem0, sem1)
    wid = lax.axis_index("s") * 2 + lax.axis_index("c")
    base = wid * _PAIRS_PER_W
    pltpu.sync_copy(labels_hbm.at[pl.ds(base, _PAIRS_PER_W)], idx_v)

    lane = lax.iota(jnp.int32, 16)
    cols = [lane + 16 * c for c in range(_NUM_TAGS // 16)]

    def build(buf, chunk):
        def fsub(sub, carry):
            labs_lo = idx_v[chunk * _CH + sub, pl.ds(0, 16)]
            labs_hi = idx_v[chunk * _CH + sub, pl.ds(_K - 16, 16)]
            for r in range(_K):
                lab = labs_lo[r] if r < 16 else labs_hi[r - (_K - 16)]
                for c in range(_NUM_TAGS // 16):
                    vals = jnp.where(cols[c] == lab, _HIGH, _LOW)
                    buf[sub, r, pl.ds(c * 16, 16)] = vals
            return carry

        lax.fori_loop(0, _CH, fsub, 0)

    def store_desc(b, chunk):
        dst = out_hbm.at[pl.ds(base + chunk * _CH, _CH)]
        return pltpu.make_async_copy(bufs[b], dst, sems[b])

    def step(g, carry):
        for b in range(_NB):
            j = g * _NB + b

            @pl.when(g >= 1)
            def _wait_prev():
                store_desc(b, j - _NB).wait()

            build(bufs[b], j)
            store_desc(b, j).start()
        return carry

    lax.fori_loop(0, _NCH // _NB, step, 0)
    for b in range(_NB):
        store_desc(b, _NCH - _NB + b).wait()


def _score_expand_tc_body(labels_ref, out_ref):
    labs = labels_ref[...]
    tags = lax.broadcasted_iota(jnp.int32, (_TC_G, _K, _NUM_TAGS), 2)
    out_ref[...] = jnp.where(tags == labs[:, :, None], _HIGH, _LOW)


_score_expand_tc = pl.pallas_call(
    _score_expand_tc_body,
    out_shape=jax.ShapeDtypeStruct((_TC_PAIRS, _K, _NUM_TAGS), jnp.float32),
    grid=(_TC_PAIRS // _TC_G,),
    in_specs=[pl.BlockSpec((_TC_G, _K), lambda i: (i, 0))],
    out_specs=pl.BlockSpec((_TC_G, _K, _NUM_TAGS), lambda i: (i, 0, 0)),
)


def kernel(arg_labels):
    labels = arg_labels.astype(jnp.int32).reshape(_NPAIR, _K)
    tc_out = _score_expand_tc(labels[:_TC_PAIRS])
    sc_out = _score_expand_sc(labels[_TC_PAIRS:])
    out = jnp.concatenate([tc_out, sc_out], axis=0)
    return out.reshape(_B, _S, _K, _NUM_TAGS)


# speedup vs baseline: 1.6516x; 1.6516x over previous
import functools

import jax
import jax.numpy as jnp
from jax import lax
from jax.experimental import pallas as pl
from jax.experimental.pallas import tpu as pltpu
from jax.experimental.pallas import tpu_sc as plsc

_NUM_TAGS = 64
_HIGH = 5.0
_LOW = -5.0

_B, _S, _K = 256, 40, 30
_NPAIR = _B * _S

_NW = 32
_PAIRS_PER_W = 48
_SC_PAIRS = _NW * _PAIRS_PER_W
_CH = 8
_NCH = _PAIRS_PER_W // _CH
_NB = 2

_TC_PAIRS = _NPAIR - _SC_PAIRS
_TC_G = 512


@functools.partial(
    pl.kernel,
    out_type=jax.ShapeDtypeStruct((_NPAIR, _K, _NUM_TAGS), jnp.float32),
    mesh=plsc.VectorSubcoreMesh(core_axis_name="c", subcore_axis_name="s"),
    scratch_types=[
        pltpu.VMEM((_PAIRS_PER_W, _K), jnp.int32),
        pltpu.VMEM((_CH, _K, _NUM_TAGS), jnp.float32),
        pltpu.VMEM((_CH, _K, _NUM_TAGS), jnp.float32),
        pltpu.SemaphoreType.DMA,
        pltpu.SemaphoreType.DMA,
    ],
)
def _score_expand_sc(labels_hbm, out_hbm, idx_v, buf0, buf1, sem0, sem1):
    bufs = (buf0, buf1)
    sems = (sem0, sem1)
    wid = lax.axis_index("s") * 2 + lax.axis_index("c")
    base = wid * _PAIRS_PER_W
    pltpu.sync_copy(labels_hbm.at[pl.ds(base, _PAIRS_PER_W)], idx_v)

    lane = lax.iota(jnp.int32, 16)
    cols = [lane + 16 * c for c in range(_NUM_TAGS // 16)]

    def build(buf, chunk):
        def fsub(sub, carry):
            labs_lo = idx_v[chunk * _CH + sub, pl.ds(0, 16)]
            labs_hi = idx_v[chunk * _CH + sub, pl.ds(_K - 16, 16)]
            for r in range(_K):
                lab = labs_lo[r] if r < 16 else labs_hi[r - (_K - 16)]
                for c in range(_NUM_TAGS // 16):
                    vals = jnp.where(cols[c] == lab, _HIGH, _LOW)
                    buf[sub, r, pl.ds(c * 16, 16)] = vals
            return carry

        lax.fori_loop(0, _CH, fsub, 0)

    def store_desc(b, chunk):
        dst = out_hbm.at[pl.ds(_TC_PAIRS + base + chunk * _CH, _CH)]
        return pltpu.make_async_copy(bufs[b], dst, sems[b])

    def step(g, carry):
        for b in range(_NB):
            j = g * _NB + b

            @pl.when(g >= 1)
            def _wait_prev():
                store_desc(b, j - _NB).wait()

            build(bufs[b], j)
            store_desc(b, j).start()
        return carry

    lax.fori_loop(0, _NCH // _NB, step, 0)
    for b in range(_NB):
        store_desc(b, _NCH - _NB + b).wait()


def _score_expand_tc_body(labels_ref, scfull_ref, out_ref):
    del scfull_ref
    labs = labels_ref[...]
    tags = lax.broadcasted_iota(jnp.int32, (_TC_G, _K, _NUM_TAGS), 2)
    out_ref[...] = jnp.where(tags == labs[:, :, None], _HIGH, _LOW)


_score_expand_tc = pl.pallas_call(
    _score_expand_tc_body,
    out_shape=jax.ShapeDtypeStruct((_NPAIR, _K, _NUM_TAGS), jnp.float32),
    grid=(_TC_PAIRS // _TC_G,),
    in_specs=[
        pl.BlockSpec((_TC_G, _K), lambda i: (i, 0)),
        pl.BlockSpec(memory_space=pl.ANY),
    ],
    out_specs=pl.BlockSpec((_TC_G, _K, _NUM_TAGS), lambda i: (i, 0, 0)),
    input_output_aliases={1: 0},
)


def kernel(arg_labels):
    labels = arg_labels.astype(jnp.int32).reshape(_NPAIR, _K)
    sc_out = _score_expand_sc(labels[_TC_PAIRS:])
    out = _score_expand_tc(labels[:_TC_PAIRS], sc_out)
    return out.reshape(_B, _S, _K, _NUM_TAGS)


# speedup vs baseline: 1.6568x; 1.0031x over previous
import jax
import jax.numpy as jnp
from jax import lax
from jax.experimental import pallas as pl
from jax.experimental.pallas import tpu as pltpu

_NUM_TAGS = 64
_HIGH = 5.0
_LOW = -5.0

_B, _S, _K = 256, 40, 30
_NPAIR = _B * _S
_G = 256
_NSTEP = _NPAIR // _G
_NB = 2


def _score_expand_body(labels_ref, out_ref, scratch, sems):
    i = pl.program_id(0)
    slot = lax.rem(i, _NB)

    def copy_step(j, s):
        return pltpu.make_async_copy(
            scratch.at[s], out_ref.at[pl.ds(j * _G, _G)], sems.at[s]
        )

    @pl.when(i >= _NB)
    def _wait_ring():
        copy_step(i - _NB, slot).wait()

    tags = lax.broadcasted_iota(jnp.int32, (_G, _K, _NUM_TAGS), 2)
    scratch[slot] = jnp.where(
        tags == labels_ref[...][:, :, None], _HIGH, _LOW
    )
    copy_step(i, slot).start()

    @pl.when(i == _NSTEP - 1)
    def _drain():
        copy_step(i - 1, lax.rem(i - 1, _NB)).wait()
        copy_step(i, slot).wait()


_score_expand = pl.pallas_call(
    _score_expand_body,
    out_shape=jax.ShapeDtypeStruct((_NPAIR, _K, _NUM_TAGS), jnp.float32),
    grid=(_NSTEP,),
    in_specs=[pl.BlockSpec((_G, _K), lambda i: (i, 0))],
    out_specs=pl.BlockSpec(memory_space=pl.ANY),
    scratch_shapes=[
        pltpu.VMEM((_NB, _G, _K, _NUM_TAGS), jnp.float32),
        pltpu.SemaphoreType.DMA((_NB,)),
    ],
)


def kernel(arg_labels):
    labels = arg_labels.astype(jnp.int32).reshape(_NPAIR, _K)
    out = _score_expand(labels)
    return out.reshape(_B, _S, _K, _NUM_TAGS)


# speedup vs baseline: 1.7149x; 1.0351x over previous
import jax
import jax.numpy as jnp
from jax import lax
from jax.experimental import pallas as pl

_NUM_TAGS = 64
_HIGH = 5.0
_LOW = -5.0

_B, _S, _K = 256, 40, 30
_NPAIR = _B * _S
_G = 512
_NSTEP = _NPAIR // _G


def _score_expand_body(labels_ref, out_ref):
    tags = lax.broadcasted_iota(jnp.int32, (_G, _K, _NUM_TAGS), 2)
    out_ref[...] = jnp.where(
        tags == labels_ref[...][:, :, None], _HIGH, _LOW
    )


_score_expand = pl.pallas_call(
    _score_expand_body,
    out_shape=jax.ShapeDtypeStruct((_NPAIR, _K, _NUM_TAGS), jnp.float32),
    grid=(_NSTEP,),
    in_specs=[pl.BlockSpec((_G, _K), lambda i: (i, 0))],
    out_specs=pl.BlockSpec((_G, _K, _NUM_TAGS), lambda i: (i, 0, 0)),
)


def kernel(arg_labels):
    labels = arg_labels.astype(jnp.int32).reshape(_NPAIR, _K)
    out = _score_expand(labels)
    return out.reshape(_B, _S, _K, _NUM_TAGS)
